# baseline (speedup 1.0000x reference)
import functools

import jax
import jax.numpy as jnp
from jax import lax
from jax.experimental import pallas as pl
from jax.experimental.pallas import tpu as pltpu
from jax.experimental.pallas import tpu_sc as plsc

LANES = 16
NC, NS = 2, 16
NW = NC * NS


def _make_sc_gather_sum(n: int, d: int):
    vpr = d // LANES
    main = (n // (NW * 8)) * 8
    tail = n - main * NW
    blk = 8
    for cand in range(128, 0, -8):
        if main % cand == 0:
            blk = cand
            break
    steps = main // blk
    mesh = plsc.VectorSubcoreMesh(core_axis_name="c", subcore_axis_name="s")

    @functools.partial(
        pl.kernel,
        mesh=mesh,
        out_type=jax.ShapeDtypeStruct((NW, d), jnp.float32),
        scratch_types=[
            pltpu.VMEM((main,), jnp.int32),
            pltpu.VMEM((blk, d), jnp.float32),
            pltpu.VMEM((blk, d), jnp.float32),
            pltpu.VMEM((max(tail, 8),), jnp.int32),
            pltpu.VMEM((d,), jnp.float32),
            pltpu.SemaphoreType.DMA,
            pltpu.SemaphoreType.DMA,
        ],
    )
    def gather_sum(table_hbm, idx_hbm, out_hbm,
                   idx_v, rows_a, rows_b, idx_t, acc_v, sem, sem_t):
        wid = lax.axis_index("s") * NC + lax.axis_index("c")
        base = wid * main
        idx_cp = pltpu.async_copy(idx_hbm.at[pl.ds(base, main)], idx_v,
                                  sem_t)

        zero = jnp.zeros((LANES,), jnp.float32)
        bufs = (rows_a, rows_b)

        @plsc.parallel_loop(0, blk, step=1)
        def _(r):
            for j in range(vpr):
                rows_a[r, pl.ds(j * LANES, LANES)] = zero
                rows_b[r, pl.ds(j * LANES, LANES)] = zero

        idx_cp.wait()
        cps = [
            pltpu.async_copy(
                table_hbm.at[idx_v.at[pl.ds(g * blk, blk)]],
                bufs[g % 2], sem, add=True)
            for g in range(steps)
        ]
        if tail:
            @pl.when(wid == NW - 1)
            def _():
                pltpu.sync_copy(idx_hbm.at[pl.ds(NW * main, tail)],
                                idx_t.at[pl.ds(0, tail)])
                pltpu.async_copy(table_hbm.at[idx_t.at[pl.ds(0, tail)]],
                                 rows_a.at[pl.ds(0, tail)], sem_t,
                                 add=True).wait()
        for cp in cps:
            cp.wait()

        def accum(rows, nrows, acc, unroll=4):
            def body(i, a):
                for u in range(unroll):
                    r = i * unroll + u
                    a = tuple(a[j] + rows[r, pl.ds(j * LANES, LANES)]
                              for j in range(vpr))
                return a
            assert nrows % unroll == 0
            return lax.fori_loop(0, nrows // unroll, body, acc)

        acc = tuple(zero for _ in range(vpr))
        acc = accum(rows_a, blk, acc)
        acc = accum(rows_b, blk, acc)

        for j in range(vpr):
            acc_v[pl.ds(j * LANES, LANES)] = acc[j]
        pltpu.sync_copy(acc_v, out_hbm.at[wid])

    return gather_sum


def _tc_finish(partials, w, bias2d, n):
    def body(p_ref, w_ref, b_ref, o_ref):
        pw = jnp.dot(p_ref[...], w_ref[...], preferred_element_type=jnp.float32)
        o_ref[...] = (jnp.sum(pw, axis=0, keepdims=True)
                      + jnp.float32(n) * b_ref[...])

    return pl.pallas_call(
        body,
        out_shape=jax.ShapeDtypeStruct((1, partials.shape[1]), jnp.float32),
    )(partials, w, bias2d)


def kernel(medicine_it, m_embeddings, W, bias):
    n = medicine_it.shape[0]
    d = m_embeddings.shape[1]
    partials = _make_sc_gather_sum(n, d)(m_embeddings, medicine_it)
    out = _tc_finish(partials, W, bias.reshape(1, d), n)
    return out.reshape(1, 1, d)


# speedup vs baseline: 1.0411x; 1.0411x over previous
import functools

import jax
import jax.numpy as jnp
from jax import lax
from jax.experimental import pallas as pl
from jax.experimental.pallas import tpu as pltpu
from jax.experimental.pallas import tpu_sc as plsc

LANES = 16
NC, NS = 2, 16
NW = NC * NS


def _make_sc_gather_sum(n: int, d: int):
    vpr = d // LANES
    main = (n // (NW * 8)) * 8
    tail = n - main * NW
    blk = 8
    for cand in range(128, 0, -8):
        if main % cand == 0:
            blk = cand
            break
    steps = main // blk
    mesh = plsc.VectorSubcoreMesh(core_axis_name="c", subcore_axis_name="s")

    @functools.partial(
        pl.kernel,
        mesh=mesh,
        out_type=jax.ShapeDtypeStruct((NW, d), jnp.float32),
        scratch_types=[
            pltpu.VMEM((main,), jnp.int32),
            pltpu.VMEM((blk, d), jnp.float32),
            pltpu.VMEM((max(tail, 8),), jnp.int32),
            pltpu.VMEM((d,), jnp.float32),
            pltpu.SemaphoreType.DMA,
            pltpu.SemaphoreType.DMA,
        ],
    )
    def gather_sum(table_hbm, idx_hbm, out_hbm,
                   idx_v, rows, idx_t, acc_v, sem, sem_t):
        wid = lax.axis_index("s") * NC + lax.axis_index("c")
        base = wid * main
        idx_cp = pltpu.async_copy(idx_hbm.at[pl.ds(base, main)], idx_v,
                                  sem_t)

        zero = jnp.zeros((LANES,), jnp.float32)

        @plsc.parallel_loop(0, blk, step=1)
        def _(r):
            for j in range(vpr):
                rows[r, pl.ds(j * LANES, LANES)] = zero

        idx_cp.wait()
        cps = [
            pltpu.async_copy(
                table_hbm.at[idx_v.at[pl.ds(g * blk, blk)]],
                rows, sem, add=True)
            for g in range(steps)
        ]
        if tail:
            @pl.when(wid == NW - 1)
            def _():
                pltpu.sync_copy(idx_hbm.at[pl.ds(NW * main, tail)],
                                idx_t.at[pl.ds(0, tail)])
                pltpu.async_copy(table_hbm.at[idx_t.at[pl.ds(0, tail)]],
                                 rows.at[pl.ds(0, tail)], sem_t,
                                 add=True).wait()
        for cp in cps:
            cp.wait()

        def accum(rows, nrows, acc, unroll=4):
            def body(i, a):
                for u in range(unroll):
                    r = i * unroll + u
                    a = tuple(a[j] + rows[r, pl.ds(j * LANES, LANES)]
                              for j in range(vpr))
                return a
            assert nrows % unroll == 0
            return lax.fori_loop(0, nrows // unroll, body, acc)

        acc = accum(rows, blk, tuple(zero for _ in range(vpr)))

        for j in range(vpr):
            acc_v[pl.ds(j * LANES, LANES)] = acc[j]
        pltpu.sync_copy(acc_v, out_hbm.at[wid])

    return gather_sum


def _tc_finish(partials, w, bias2d, n):
    def body(p_ref, w_ref, b_ref, o_ref):
        pw = jnp.dot(p_ref[...], w_ref[...], preferred_element_type=jnp.float32)
        o_ref[...] = (jnp.sum(pw, axis=0, keepdims=True)
                      + jnp.float32(n) * b_ref[...])

    return pl.pallas_call(
        body,
        out_shape=jax.ShapeDtypeStruct((1, partials.shape[1]), jnp.float32),
    )(partials, w, bias2d)


def kernel(medicine_it, m_embeddings, W, bias):
    n = medicine_it.shape[0]
    d = m_embeddings.shape[1]
    partials = _make_sc_gather_sum(n, d)(m_embeddings, medicine_it)
    out = _tc_finish(partials, W, bias.reshape(1, d), n)
    return out.reshape(1, 1, d)


# speedup vs baseline: 1.0435x; 1.0022x over previous
import functools

import jax
import jax.numpy as jnp
from jax import lax
from jax.experimental import pallas as pl
from jax.experimental.pallas import tpu as pltpu
from jax.experimental.pallas import tpu_sc as plsc

LANES = 16
NC, NS = 2, 16
NW = NC * NS


def _make_sc_gather_sum(n: int, d: int):
    vpr = d // LANES
    main = (n // (NW * 8)) * 8
    tail = n - main * NW
    blk = 8
    for cand in range(128, 0, -8):
        if main % cand == 0:
            blk = cand
            break
    steps = main // blk
    mesh = plsc.VectorSubcoreMesh(core_axis_name="c", subcore_axis_name="s")

    @functools.partial(
        pl.kernel,
        mesh=mesh,
        out_type=jax.ShapeDtypeStruct((NW, d), jnp.float32),
        scratch_types=[
            pltpu.VMEM((main,), jnp.int32),
            pltpu.VMEM((blk, d), jnp.float32),
            pltpu.VMEM((max(tail, 8),), jnp.int32),
            pltpu.VMEM((d,), jnp.float32),
            pltpu.SemaphoreType.DMA,
            pltpu.SemaphoreType.DMA,
        ],
    )
    def gather_sum(table_hbm, idx_hbm, out_hbm,
                   idx_v, rows, idx_t, acc_v, sem, sem_t):
        wid = lax.axis_index("s") * NC + lax.axis_index("c")
        base = wid * main
        idx_cp = pltpu.async_copy(idx_hbm.at[pl.ds(base, main)], idx_v,
                                  sem_t)

        zero = jnp.zeros((LANES,), jnp.float32)

        @plsc.parallel_loop(0, blk, step=1)
        def _(r):
            for j in range(vpr):
                rows[r, pl.ds(j * LANES, LANES)] = zero

        idx_cp.wait()
        cps = [
            pltpu.async_copy(
                table_hbm.at[idx_v.at[pl.ds(g * blk, blk)]],
                rows, sem, add=True)
            for g in range(steps)
        ]
        if tail:
            assert tail % 8 == 0
            ntw = tail // 8

            @pl.when(wid < ntw)
            def _():
                pltpu.sync_copy(
                    idx_hbm.at[pl.ds(NW * main + wid * 8, 8)],
                    idx_t.at[pl.ds(0, 8)])
                pltpu.async_copy(table_hbm.at[idx_t.at[pl.ds(0, 8)]],
                                 rows.at[pl.ds(0, 8)], sem_t,
                                 add=True).wait()
        for cp in cps:
            cp.wait()

        def accum(rows, nrows, acc, unroll=4):
            def body(i, a):
                for u in range(unroll):
                    r = i * unroll + u
                    a = tuple(a[j] + rows[r, pl.ds(j * LANES, LANES)]
                              for j in range(vpr))
                return a
            assert nrows % unroll == 0
            return lax.fori_loop(0, nrows // unroll, body, acc)

        acc = accum(rows, blk, tuple(zero for _ in range(vpr)))

        for j in range(vpr):
            acc_v[pl.ds(j * LANES, LANES)] = acc[j]
        pltpu.sync_copy(acc_v, out_hbm.at[wid])

    return gather_sum


def _tc_finish(partials, w, bias2d, n):
    def body(p_ref, w_ref, b_ref, o_ref):
        pw = jnp.dot(p_ref[...], w_ref[...], preferred_element_type=jnp.float32)
        o_ref[...] = (jnp.sum(pw, axis=0, keepdims=True)
                      + jnp.float32(n) * b_ref[...])

    return pl.pallas_call(
        body,
        out_shape=jax.ShapeDtypeStruct((1, partials.shape[1]), jnp.float32),
    )(partials, w, bias2d)


def kernel(medicine_it, m_embeddings, W, bias):
    n = medicine_it.shape[0]
    d = m_embeddings.shape[1]
    partials = _make_sc_gather_sum(n, d)(m_embeddings, medicine_it)
    out = _tc_finish(partials, W, bias.reshape(1, d), n)
    return out.reshape(1, 1, d)
